# baseline (speedup 1.0000x reference)
import functools

import jax
import jax.numpy as jnp
from jax import lax
from jax.experimental import pallas as pl
from jax.experimental.pallas import tpu as pltpu
from jax.experimental.pallas import tpu_sc as plsc

G_SEL = 50000
EMBED_DIM = 64

NC, NS = 2, 16
NW = NC * NS
BK = 2048
K_PAD = 51200
BPW = K_PAD // NW
SUB = 100
NSUB = BPW // SUB

_mesh = plsc.VectorSubcoreMesh(core_axis_name="c", subcore_axis_name="s")


@functools.partial(
    pl.kernel,
    mesh=_mesh,
    out_type=jax.ShapeDtypeStruct((K_PAD, EMBED_DIM), jnp.float32),
    scratch_types=[
        pltpu.VMEM((NSUB, SUB), jnp.int32),
        pltpu.VMEM((BPW, EMBED_DIM), jnp.float32),
        pltpu.SemaphoreType.DMA,
    ],
    compiler_params=pltpu.CompilerParams(use_tc_tiling_on_sc=False),
)
def _sc_gather(table_hbm, idx_hbm, out_hbm, idx_v, rows_v, sem):
    wid = lax.axis_index("s") * NC + lax.axis_index("c")
    base = wid * BPW
    pltpu.sync_copy(idx_hbm.at[pl.ds(wid * NSUB, NSUB)], idx_v)
    copies = [
        pltpu.async_copy(
            table_hbm.at[idx_v.at[j]],
            rows_v.at[pl.ds(j * SUB, SUB)],
            sem,
        )
        for j in range(NSUB)
    ]
    for c in copies:
        c.wait()
    pltpu.sync_copy(rows_v, out_hbm.at[pl.ds(base, BPW)])


def _mm_body(x_ref, g_ref, o_ref):
    k = pl.program_id(0)

    @pl.when(k == 0)
    def _():
        o_ref[...] = jnp.zeros_like(o_ref)

    xb = x_ref[...]
    col = k * BK + lax.broadcasted_iota(jnp.int32, (1, BK), 1)
    xb = jnp.where(col < G_SEL, xb, 0.0)
    o_ref[...] += jnp.dot(xb, g_ref[...], preferred_element_type=jnp.float32)


def _tc_matmul(x, g):
    grid = K_PAD // BK
    return pl.pallas_call(
        _mm_body,
        grid=(grid,),
        in_specs=[
            pl.BlockSpec((x.shape[0], BK), lambda k: (0, k)),
            pl.BlockSpec((BK, EMBED_DIM), lambda k: (k, 0)),
        ],
        out_specs=pl.BlockSpec((x.shape[0], EMBED_DIM), lambda k: (0, 0)),
        out_shape=jax.ShapeDtypeStruct((x.shape[0], EMBED_DIM), jnp.float32),
        compiler_params=pltpu.CompilerParams(
            dimension_semantics=("arbitrary",),
        ),
    )(x, g)


def kernel(x, gene_idx, gene_embeddings):
    idx_pad = jnp.concatenate(
        [gene_idx, jnp.zeros((K_PAD - G_SEL,), jnp.int32)]
    ).reshape(K_PAD // SUB, SUB)
    g = _sc_gather(gene_embeddings, idx_pad)
    return _tc_matmul(x, g)
